# Initial kernel scaffold; baseline (speedup 1.0000x reference)
import functools

import jax
import jax.numpy as jnp
from jax import lax
from jax.experimental import pallas as pl
from jax.experimental.pallas import tpu as pltpu
from jax.experimental.pallas import tpu_sc as plsc

N = 10000
E = 320000
C_IN = 128
HID = 256
C_OUT = 128

NC = 2
NS = 16
NW = NC * NS

NPW = 320
NPAD = NW * NPW
RPT_LEN = NPW + 8
RPT_PAD = (NW - 1) * NPW + RPT_LEN
CH = 128
EPAD = E + CH

GA = -2.0 * 0.7978845608028654
GB = GA * 0.044715

VB = HID // 16


def _mm_ps_body(x_ref, w_ref, b1_ref, p_ref, s_ref):
    ps = jnp.dot(x_ref[...], w_ref[...], preferred_element_type=jnp.float32)
    p_ref[...] = ps[:, :HID]
    s_ref[...] = ps[:, HID:] + b1_ref[...]


def _mm_out_body(g_ref, w2_ref, b2_ref, rhi_ref, rlo_ref, o_ref):
    cnt = (rhi_ref[0, 0, :] - rlo_ref[0, 0, :]).astype(jnp.float32)
    scale = 1.0 / jnp.maximum(cnt, 1.0)
    gs = g_ref[...] * scale[:, None]
    y = jnp.dot(gs, w2_ref[...], preferred_element_type=jnp.float32)
    o_ref[...] = y + b2_ref[...] * (cnt > 0.0).astype(jnp.float32)[:, None]


def _sc_segment_gelu(p_hbm, s_hbm, idx_hbm, rpt_hbm, g_hbm,
                     rpt_v, idx_v, rows_v, srow_v, acc_v, sem):
    c = lax.axis_index("c")
    s = lax.axis_index("s")
    wid = s * NC + c
    n0 = wid * NPW

    pltpu.sync_copy(rpt_hbm.at[pl.ds(n0, RPT_LEN)], rpt_v)
    e_start = rpt_v[0]
    e_end = rpt_v[NPW]

    zeros16 = jnp.zeros((16,), jnp.float32)

    def load_srow(node):
        pltpu.sync_copy(s_hbm.at[node], srow_v)
        return tuple(srow_v[pl.ds(j * 16, 16)] for j in range(VB))

    def flush(i, acc):
        for j in range(VB):
            acc_v[pl.ds(j * 16, 16)] = acc[j]
        pltpu.sync_copy(acc_v, g_hbm.at[n0 + i])

    srow0 = load_srow(n0)
    acc0 = (zeros16,) * VB

    def chunk_cond(st):
        return st[0] < e_end

    def chunk_body(st):
        a0, i, e1, srow, acc = st
        pltpu.sync_copy(idx_hbm.at[pl.ds(a0, CH)], idx_v)
        pltpu.async_copy(p_hbm.at[idx_v], rows_v, sem).wait()
        lo = jnp.maximum(e_start, a0) - a0
        hi = jnp.minimum(e_end, a0 + CH) - a0

        def edge_body(r, est):
            i_, e1_, srow_, acc_ = est
            g = a0 + r

            def adv_cond(ast):
                return g >= ast[1]

            def adv_body(ast):
                ai, _, asrow, aacc = ast
                flush(ai, aacc)
                ai = ai + 1
                return (ai, rpt_v[ai + 1], load_srow(n0 + ai),
                        (zeros16,) * VB)

            i_, e1_, srow_, acc_ = lax.while_loop(
                adv_cond, adv_body, (i_, e1_, srow_, acc_))

            new_acc = []
            for j in range(VB):
                x = rows_v[r, pl.ds(j * 16, 16)] + srow_[j]
                z = x * (GA + GB * (x * x))
                new_acc.append(acc_[j] + x / (1.0 + jnp.exp(z)))
            return (i_, e1_, srow_, tuple(new_acc))

        i, e1, srow, acc = lax.fori_loop(lo, hi, edge_body,
                                         (i, e1, srow, acc))
        return (a0 + CH, i, e1, srow, acc)

    a0_init = (e_start // 8) * 8
    _, i_fin, _, _, acc_fin = lax.while_loop(
        chunk_cond, chunk_body, (a0_init, 0, rpt_v[1], srow0, acc0))

    def tail_cond(st):
        return st[0] < NPW

    def tail_body(st):
        ti, tacc = st
        flush(ti, tacc)
        return (ti + 1, (zeros16,) * VB)

    lax.while_loop(tail_cond, tail_body, (i_fin, acc_fin))


@functools.partial(
    pl.kernel,
    mesh=plsc.VectorSubcoreMesh(core_axis_name="c", subcore_axis_name="s"),
    out_type=jax.ShapeDtypeStruct((NPAD, HID), jnp.float32),
    scratch_types=[
        pltpu.VMEM((RPT_LEN,), jnp.int32),
        pltpu.VMEM((CH,), jnp.int32),
        pltpu.VMEM((CH, HID), jnp.float32),
        pltpu.VMEM((HID,), jnp.float32),
        pltpu.VMEM((HID,), jnp.float32),
        pltpu.SemaphoreType.DMA,
    ],
)
def _sc_kernel(p_hbm, s_hbm, idx_hbm, rpt_hbm, g_hbm,
               rpt_v, idx_v, rows_v, srow_v, acc_v, sem):
    _sc_segment_gelu(p_hbm, s_hbm, idx_hbm, rpt_hbm, g_hbm,
                     rpt_v, idx_v, rows_v, srow_v, acc_v, sem)


def kernel(in_features, W1, b1, W2, b2, neighbor_idx, rowptr):
    x = in_features[0]
    xp = jnp.pad(x, ((0, NPAD - N), (0, 0)))
    wc = jnp.concatenate([W1[:C_IN], W1[C_IN:]], axis=1)
    b1r = b1.reshape(1, HID)

    nblk = NPAD // 512
    p_arr, s_arr = pl.pallas_call(
        _mm_ps_body,
        grid=(nblk,),
        in_specs=[
            pl.BlockSpec((512, C_IN), lambda i: (i, 0)),
            pl.BlockSpec((C_IN, 2 * HID), lambda i: (0, 0)),
            pl.BlockSpec((1, HID), lambda i: (0, 0)),
        ],
        out_specs=[
            pl.BlockSpec((512, HID), lambda i: (i, 0)),
            pl.BlockSpec((512, HID), lambda i: (i, 0)),
        ],
        out_shape=[
            jax.ShapeDtypeStruct((NPAD, HID), jnp.float32),
            jax.ShapeDtypeStruct((NPAD, HID), jnp.float32),
        ],
    )(xp, wc, b1r)

    idx32 = neighbor_idx.astype(jnp.int32)
    rpt32 = rowptr.astype(jnp.int32)
    idxp = jnp.pad(idx32, (0, EPAD - E))
    rptp = jnp.pad(rpt32, (0, RPT_PAD - (N + 1)), constant_values=E)

    g_arr = _sc_kernel(p_arr, s_arr, idxp, rptp)

    rhi = rptp[1:NPAD + 1].reshape(nblk, 1, 512)
    rlo = rptp[:NPAD].reshape(nblk, 1, 512)
    b2r = b2.reshape(1, C_OUT)

    out = pl.pallas_call(
        _mm_out_body,
        grid=(nblk,),
        in_specs=[
            pl.BlockSpec((512, HID), lambda i: (i, 0)),
            pl.BlockSpec((HID, C_OUT), lambda i: (0, 0)),
            pl.BlockSpec((1, C_OUT), lambda i: (0, 0)),
            pl.BlockSpec((1, 1, 512), lambda i: (i, 0, 0)),
            pl.BlockSpec((1, 1, 512), lambda i: (i, 0, 0)),
        ],
        out_specs=pl.BlockSpec((512, C_OUT), lambda i: (i, 0)),
        out_shape=jax.ShapeDtypeStruct((NPAD, C_OUT), jnp.float32),
    )(g_arr, W2, b2r, rhi, rlo)

    return out[:N].reshape(1, N, C_OUT)


# speedup vs baseline: 3.7580x; 3.7580x over previous
import functools

import jax
import jax.numpy as jnp
from jax import lax
from jax.experimental import pallas as pl
from jax.experimental.pallas import tpu as pltpu
from jax.experimental.pallas import tpu_sc as plsc

N = 10000
E = 320000
C_IN = 128
HID = 256
C_OUT = 128

NC = 2
NS = 16
NW = NC * NS

NPW = 320
NPAD = NW * NPW
RPT_LEN = NPW + 16
RPT_PAD = (NW - 1) * NPW + RPT_LEN
CH = 64
EPAD = E + CH

GA = -2.0 * 0.7978845608028654
GB = GA * 0.044715

VB = HID // 16


def _mm_ps_body(x_ref, w_ref, b1_ref, p_ref, s_ref):
    ps = jnp.dot(x_ref[...], w_ref[...], preferred_element_type=jnp.float32)
    p_ref[...] = ps[:, :HID]
    s_ref[...] = ps[:, HID:] + b1_ref[...]


def _mm_out_body(g_ref, w2_ref, b2_ref, rhi_ref, rlo_ref, o_ref):
    cnt = (rhi_ref[0, 0, :] - rlo_ref[0, 0, :]).astype(jnp.float32)
    scale = 1.0 / jnp.maximum(cnt, 1.0)
    gs = g_ref[...] * scale[:, None]
    y = jnp.dot(gs, w2_ref[...], preferred_element_type=jnp.float32)
    o_ref[...] = y + b2_ref[...] * (cnt > 0.0).astype(jnp.float32)[:, None]


def _sc_segment_gelu(p_hbm, s_hbm, idx_hbm, rpt_hbm, g_hbm,
                     rpt_v, idx_v, rows_v, srow_v, acc_v, sem, sem2):
    c = lax.axis_index("c")
    s = lax.axis_index("s")
    wid = s * NC + c
    n0 = wid * NPW

    pltpu.async_copy(rpt_hbm.at[pl.ds(pl.multiple_of(n0, 8), RPT_LEN)],
                     rpt_v, sem2).wait()

    def rv(k):
        return rpt_v[pl.ds(k, 16)][0]

    zeros16 = jnp.zeros((16,), jnp.float32)

    def node_body(i, _):
        e0 = rv(i)
        e1 = rv(i + 1)
        pltpu.async_copy(s_hbm.at[n0 + i], srow_v, sem2).wait()
        srow = tuple(srow_v[pl.ds(j * 16, 16)] for j in range(VB))
        a0 = (e0 // 8) * 8
        nch = (e1 - a0 + CH - 1) // CH

        def chunk_body(k, acc):
            a = a0 + k * CH
            pltpu.async_copy(idx_hbm.at[pl.ds(pl.multiple_of(a, 8), CH)],
                             idx_v, sem2).wait()
            pltpu.async_copy(p_hbm.at[idx_v], rows_v, sem).wait()
            lo = jnp.maximum(e0, a) - a
            hi = jnp.minimum(e1, a + CH) - a

            def edge_body(r, acc_):
                new_acc = []
                for j in range(VB):
                    x = rows_v[r, pl.ds(j * 16, 16)] + srow[j]
                    z = x * (GA + GB * (x * x))
                    new_acc.append(acc_[j] + x / (1.0 + jnp.exp(z)))
                return tuple(new_acc)

            return lax.fori_loop(lo, hi, edge_body, acc)

        acc = lax.fori_loop(0, nch, chunk_body, (zeros16,) * VB)
        for j in range(VB):
            acc_v[pl.ds(j * 16, 16)] = acc[j]
        pltpu.async_copy(acc_v, g_hbm.at[n0 + i], sem2).wait()
        return 0

    lax.fori_loop(0, NPW, node_body, 0)


@functools.partial(
    pl.kernel,
    mesh=plsc.VectorSubcoreMesh(core_axis_name="c", subcore_axis_name="s"),
    out_type=jax.ShapeDtypeStruct((NPAD, HID), jnp.float32),
    scratch_types=[
        pltpu.VMEM((RPT_LEN,), jnp.int32),
        pltpu.VMEM((CH,), jnp.int32),
        pltpu.VMEM((CH, HID), jnp.float32),
        pltpu.VMEM((HID,), jnp.float32),
        pltpu.VMEM((HID,), jnp.float32),
        pltpu.SemaphoreType.DMA,
        pltpu.SemaphoreType.DMA,
    ],
)
def _sc_kernel(p_hbm, s_hbm, idx_hbm, rpt_hbm, g_hbm,
               rpt_v, idx_v, rows_v, srow_v, acc_v, sem, sem2):
    _sc_segment_gelu(p_hbm, s_hbm, idx_hbm, rpt_hbm, g_hbm,
                     rpt_v, idx_v, rows_v, srow_v, acc_v, sem, sem2)


def kernel(in_features, W1, b1, W2, b2, neighbor_idx, rowptr):
    x = in_features[0]
    xp = jnp.pad(x, ((0, NPAD - N), (0, 0)))
    wc = jnp.concatenate([W1[:C_IN], W1[C_IN:]], axis=1)
    b1r = b1.reshape(1, HID)

    nblk = NPAD // 512
    p_arr, s_arr = pl.pallas_call(
        _mm_ps_body,
        grid=(nblk,),
        in_specs=[
            pl.BlockSpec((512, C_IN), lambda i: (i, 0)),
            pl.BlockSpec((C_IN, 2 * HID), lambda i: (0, 0)),
            pl.BlockSpec((1, HID), lambda i: (0, 0)),
        ],
        out_specs=[
            pl.BlockSpec((512, HID), lambda i: (i, 0)),
            pl.BlockSpec((512, HID), lambda i: (i, 0)),
        ],
        out_shape=[
            jax.ShapeDtypeStruct((NPAD, HID), jnp.float32),
            jax.ShapeDtypeStruct((NPAD, HID), jnp.float32),
        ],
    )(xp, wc, b1r)

    idx32 = neighbor_idx.astype(jnp.int32)
    rpt32 = rowptr.astype(jnp.int32)
    idxp = jnp.pad(idx32, (0, EPAD - E))
    rptp = jnp.pad(rpt32, (0, RPT_PAD - (N + 1)), constant_values=E)

    g_arr = _sc_kernel(p_arr, s_arr, idxp, rptp)

    rhi = rptp[1:NPAD + 1].reshape(nblk, 1, 512)
    rlo = rptp[:NPAD].reshape(nblk, 1, 512)
    b2r = b2.reshape(1, C_OUT)

    out = pl.pallas_call(
        _mm_out_body,
        grid=(nblk,),
        in_specs=[
            pl.BlockSpec((512, HID), lambda i: (i, 0)),
            pl.BlockSpec((HID, C_OUT), lambda i: (0, 0)),
            pl.BlockSpec((1, C_OUT), lambda i: (0, 0)),
            pl.BlockSpec((1, 1, 512), lambda i: (i, 0, 0)),
            pl.BlockSpec((1, 1, 512), lambda i: (i, 0, 0)),
        ],
        out_specs=pl.BlockSpec((512, C_OUT), lambda i: (i, 0)),
        out_shape=jax.ShapeDtypeStruct((NPAD, C_OUT), jnp.float32),
    )(g_arr, W2, b2r, rhi, rlo)

    return out[:N].reshape(1, N, C_OUT)


# speedup vs baseline: 4.2698x; 1.1362x over previous
import functools

import jax
import jax.numpy as jnp
from jax import lax
from jax.experimental import pallas as pl
from jax.experimental.pallas import tpu as pltpu
from jax.experimental.pallas import tpu_sc as plsc

N = 10000
E = 320000
C_IN = 128
HID = 256
C_OUT = 128

NC = 2
NS = 16
NW = NC * NS

NPW = 320
NPAD = NW * NPW
RPT_LEN = NPW + 16
RPT_PAD = (NW - 1) * NPW + RPT_LEN
CH = 64
FB = 16
EPAD = E + CH

GA = -2.0 * 0.7978845608028654
GB = GA * 0.044715

VB = HID // 16


def _mm_ps_body(x_ref, w_ref, b1_ref, p_ref, s_ref):
    ps = jnp.dot(x_ref[...], w_ref[...], preferred_element_type=jnp.float32)
    p_ref[...] = ps[:, :HID]
    s_ref[...] = ps[:, HID:] + b1_ref[...]


def _mm_out_body(g_ref, w2_ref, b2_ref, rhi_ref, rlo_ref, o_ref):
    cnt = (rhi_ref[0, 0, :] - rlo_ref[0, 0, :]).astype(jnp.float32)
    scale = 1.0 / jnp.maximum(cnt, 1.0)
    gs = g_ref[...] * scale[:, None]
    y = jnp.dot(gs, w2_ref[...], preferred_element_type=jnp.float32)
    o_ref[...] = y + b2_ref[...] * (cnt > 0.0).astype(jnp.float32)[:, None]


def _sc_segment_gelu(p_hbm, s_hbm, idx_hbm, rpt_hbm, g_hbm,
                     rpt_v, idx_v, rows_v, s_all, flush_v, sem, sem2):
    c = lax.axis_index("c")
    s = lax.axis_index("s")
    wid = s * NC + c
    n0 = wid * NPW

    pltpu.async_copy(rpt_hbm.at[pl.ds(pl.multiple_of(n0, 8), RPT_LEN)],
                     rpt_v, sem2).wait()
    pltpu.async_copy(s_hbm.at[pl.ds(n0, NPW)], s_all, sem2).wait()

    def rv(k):
        return rpt_v[pl.ds(k, 16)][0]

    zeros16 = jnp.zeros((16,), jnp.float32)

    def node_body(i, _):
        e0 = rv(i)
        e1 = rv(i + 1)
        srow = tuple(s_all[i, pl.ds(j * 16, 16)] for j in range(VB))
        a0 = (e0 // 8) * 8
        nch = (e1 - a0 + CH - 1) // CH

        def chunk_body(k, acc):
            a = a0 + k * CH
            pltpu.async_copy(idx_hbm.at[pl.ds(pl.multiple_of(a, 8), CH)],
                             idx_v, sem2).wait()
            pltpu.async_copy(p_hbm.at[idx_v], rows_v, sem).wait()
            lo = jnp.maximum(e0, a) - a
            hi = jnp.minimum(e1, a + CH) - a

            def edge_body(r, acc_):
                new_acc = []
                for j in range(VB):
                    x = rows_v[r, pl.ds(j * 16, 16)] + srow[j]
                    z = x * (GA + GB * (x * x))
                    new_acc.append(acc_[j] + x / (1.0 + jnp.exp(z)))
                return tuple(new_acc)

            return lax.fori_loop(lo, hi, edge_body, acc)

        acc = lax.fori_loop(0, nch, chunk_body, (zeros16,) * VB)
        for j in range(VB):
            flush_v[i % FB, pl.ds(j * 16, 16)] = acc[j]

        @pl.when(i % FB == FB - 1)
        def _():
            pltpu.async_copy(
                flush_v, g_hbm.at[pl.ds(n0 + (i // FB) * FB, FB)],
                sem2).wait()

        return 0

    lax.fori_loop(0, NPW, node_body, 0)


@functools.partial(
    pl.kernel,
    mesh=plsc.VectorSubcoreMesh(core_axis_name="c", subcore_axis_name="s"),
    out_type=jax.ShapeDtypeStruct((NPAD, HID), jnp.float32),
    scratch_types=[
        pltpu.VMEM((RPT_LEN,), jnp.int32),
        pltpu.VMEM((CH,), jnp.int32),
        pltpu.VMEM((CH, HID), jnp.float32),
        pltpu.VMEM((NPW, HID), jnp.float32),
        pltpu.VMEM((FB, HID), jnp.float32),
        pltpu.SemaphoreType.DMA,
        pltpu.SemaphoreType.DMA,
    ],
)
def _sc_kernel(p_hbm, s_hbm, idx_hbm, rpt_hbm, g_hbm,
               rpt_v, idx_v, rows_v, s_all, flush_v, sem, sem2):
    _sc_segment_gelu(p_hbm, s_hbm, idx_hbm, rpt_hbm, g_hbm,
                     rpt_v, idx_v, rows_v, s_all, flush_v, sem, sem2)


def kernel(in_features, W1, b1, W2, b2, neighbor_idx, rowptr):
    x = in_features[0]
    xp = jnp.pad(x, ((0, NPAD - N), (0, 0)))
    wc = jnp.concatenate([W1[:C_IN], W1[C_IN:]], axis=1)
    b1r = b1.reshape(1, HID)

    nblk = NPAD // 512
    p_arr, s_arr = pl.pallas_call(
        _mm_ps_body,
        grid=(nblk,),
        in_specs=[
            pl.BlockSpec((512, C_IN), lambda i: (i, 0)),
            pl.BlockSpec((C_IN, 2 * HID), lambda i: (0, 0)),
            pl.BlockSpec((1, HID), lambda i: (0, 0)),
        ],
        out_specs=[
            pl.BlockSpec((512, HID), lambda i: (i, 0)),
            pl.BlockSpec((512, HID), lambda i: (i, 0)),
        ],
        out_shape=[
            jax.ShapeDtypeStruct((NPAD, HID), jnp.float32),
            jax.ShapeDtypeStruct((NPAD, HID), jnp.float32),
        ],
    )(xp, wc, b1r)

    idx32 = neighbor_idx.astype(jnp.int32)
    rpt32 = rowptr.astype(jnp.int32)
    idxp = jnp.pad(idx32, (0, EPAD - E))
    rptp = jnp.pad(rpt32, (0, RPT_PAD - (N + 1)), constant_values=E)

    g_arr = _sc_kernel(p_arr, s_arr, idxp, rptp)

    rhi = rptp[1:NPAD + 1].reshape(nblk, 1, 512)
    rlo = rptp[:NPAD].reshape(nblk, 1, 512)
    b2r = b2.reshape(1, C_OUT)

    out = pl.pallas_call(
        _mm_out_body,
        grid=(nblk,),
        in_specs=[
            pl.BlockSpec((512, HID), lambda i: (i, 0)),
            pl.BlockSpec((HID, C_OUT), lambda i: (0, 0)),
            pl.BlockSpec((1, C_OUT), lambda i: (0, 0)),
            pl.BlockSpec((1, 1, 512), lambda i: (i, 0, 0)),
            pl.BlockSpec((1, 1, 512), lambda i: (i, 0, 0)),
        ],
        out_specs=pl.BlockSpec((512, C_OUT), lambda i: (i, 0)),
        out_shape=jax.ShapeDtypeStruct((NPAD, C_OUT), jnp.float32),
    )(g_arr, W2, b2r, rhi, rlo)

    return out[:N].reshape(1, N, C_OUT)
